# baseline (speedup 1.0000x reference)
import jax
import jax.numpy as jnp
from jax import lax
from jax.experimental import pallas as pl
from jax.experimental.pallas import tpu as pltpu
from jax.experimental.pallas import tpu_sc as plsc

_B, _L = 4096, 32
_N = _B * _L
_D0, _D1 = 2048, 1024
_NW = 32
_TPW = _N // _NW
_IW = 128
_W0, _R0 = 8, 4
_W1, _R1 = 32, 2

_mesh = plsc.VectorSubcoreMesh(core_axis_name="core", subcore_axis_name="subcore")


def _phase(table_hbm, idx_hbm, out_hbm, d, w, r_depth, wid):
    n = _TPW // w
    per_row = _IW // w
    base = wid * _TPW

    def run(idx_v, bufs, gsem, ssem):
        pltpu.sync_copy(idx_hbm.at[wid], idx_v)

        def idx_slice(g):
            return idx_v.at[g // per_row, pl.ds((g % per_row) * w, w)]

        def gather(g, slot):
            pltpu.async_copy(table_hbm.at[idx_slice(g)], bufs.at[slot], gsem.at[slot])

        def out_slice(g):
            return out_hbm.at[pl.ds(base + g * w, w)]

        for slot in range(r_depth - 1):
            gather(slot, slot)

        @pl.loop(0, n, step=r_depth)
        def _(gg):
            for r in range(r_depth):
                g = gg + r
                rm1 = (r - 1) % r_depth

                @pl.when(g >= 1)
                def _():
                    pltpu.make_async_copy(bufs.at[rm1], out_slice(g - 1), ssem.at[rm1]).wait()

                @pl.when(g + r_depth - 1 < n)
                def _():
                    gather(g + r_depth - 1, rm1)

                pltpu.make_async_copy(table_hbm.at[idx_slice(g)], bufs.at[r], gsem.at[r]).wait()
                pltpu.async_copy(bufs.at[r], out_slice(g), ssem.at[r])

        last = (n - 1) % r_depth
        pltpu.make_async_copy(bufs.at[last], out_slice(n - 1), ssem.at[last]).wait()

    pl.run_scoped(
        run,
        pltpu.VMEM((n // per_row, _IW), jnp.int32),
        pltpu.VMEM((r_depth, w, d), jnp.float32),
        pltpu.SemaphoreType.DMA((r_depth,)),
        pltpu.SemaphoreType.DMA((r_depth,)),
    )


def _embed_pair(idx0, idx1, table0, table1):
    @pl.kernel(
        out_type=(
            jax.ShapeDtypeStruct((_N, _D0), jnp.float32),
            jax.ShapeDtypeStruct((_N, _D1), jnp.float32),
        ),
        mesh=_mesh,
    )
    def body(t0_hbm, i0_hbm, t1_hbm, i1_hbm, o0_hbm, o1_hbm):
        wid = lax.axis_index("subcore") * 2 + lax.axis_index("core")
        _phase(t0_hbm, i0_hbm, o0_hbm, _D0, _W0, _R0, wid)
        _phase(t1_hbm, i1_hbm, o1_hbm, _D1, _W1, _R1, wid)

    return body(table0, idx0, table1, idx1)


def kernel(captions_0, captions_1, opt_word_embed, t5_word_embed):
    idx0 = captions_0.reshape(_NW, _TPW // _IW, _IW)
    idx1 = captions_1.reshape(_NW, _TPW // _IW, _IW)
    o0, o1 = _embed_pair(idx0, idx1, opt_word_embed, t5_word_embed)
    return o0.reshape(_B, _L, _D0), o1.reshape(_B, _L, _D1)


# speedup vs baseline: 1.0031x; 1.0031x over previous
import jax
import jax.numpy as jnp
from jax import lax
from jax.experimental import pallas as pl
from jax.experimental.pallas import tpu as pltpu
from jax.experimental.pallas import tpu_sc as plsc

_B, _L = 4096, 32
_N = _B * _L
_D0, _D1 = 2048, 1024
_NW = 32
_TPW = _N // _NW
_IW = 128
_W0, _R0 = 8, 4
_W1, _R1 = 16, 4

_mesh = plsc.VectorSubcoreMesh(core_axis_name="core", subcore_axis_name="subcore")


def _phase(table_hbm, idx_hbm, out_hbm, d, w, r_depth, wid):
    n = _TPW // w
    per_row = _IW // w
    base = wid * _TPW

    def run(idx_v, bufs, gsem, ssem):
        pltpu.sync_copy(idx_hbm.at[wid], idx_v)

        def idx_slice(g):
            return idx_v.at[g // per_row, pl.ds((g % per_row) * w, w)]

        def gather(g, slot):
            pltpu.async_copy(table_hbm.at[idx_slice(g)], bufs.at[slot], gsem.at[slot])

        def out_slice(g):
            return out_hbm.at[pl.ds(base + g * w, w)]

        for slot in range(r_depth - 1):
            gather(slot, slot)

        @pl.loop(0, n, step=r_depth)
        def _(gg):
            for r in range(r_depth):
                g = gg + r
                rm1 = (r - 1) % r_depth

                @pl.when(g >= 1)
                def _():
                    pltpu.make_async_copy(bufs.at[rm1], out_slice(g - 1), ssem.at[rm1]).wait()

                @pl.when(g + r_depth - 1 < n)
                def _():
                    gather(g + r_depth - 1, rm1)

                pltpu.make_async_copy(table_hbm.at[idx_slice(g)], bufs.at[r], gsem.at[r]).wait()
                pltpu.async_copy(bufs.at[r], out_slice(g), ssem.at[r])

        last = (n - 1) % r_depth
        pltpu.make_async_copy(bufs.at[last], out_slice(n - 1), ssem.at[last]).wait()

    pl.run_scoped(
        run,
        pltpu.VMEM((n // per_row, _IW), jnp.int32),
        pltpu.VMEM((r_depth, w, d), jnp.float32),
        pltpu.SemaphoreType.DMA((r_depth,)),
        pltpu.SemaphoreType.DMA((r_depth,)),
    )


def _embed_pair(idx0, idx1, table0, table1):
    @pl.kernel(
        out_type=(
            jax.ShapeDtypeStruct((_N, _D0), jnp.float32),
            jax.ShapeDtypeStruct((_N, _D1), jnp.float32),
        ),
        mesh=_mesh,
    )
    def body(t0_hbm, i0_hbm, t1_hbm, i1_hbm, o0_hbm, o1_hbm):
        wid = lax.axis_index("subcore") * 2 + lax.axis_index("core")
        _phase(t0_hbm, i0_hbm, o0_hbm, _D0, _W0, _R0, wid)
        _phase(t1_hbm, i1_hbm, o1_hbm, _D1, _W1, _R1, wid)

    return body(table0, idx0, table1, idx1)


def kernel(captions_0, captions_1, opt_word_embed, t5_word_embed):
    idx0 = captions_0.reshape(_NW, _TPW // _IW, _IW)
    idx1 = captions_1.reshape(_NW, _TPW // _IW, _IW)
    o0, o1 = _embed_pair(idx0, idx1, opt_word_embed, t5_word_embed)
    return o0.reshape(_B, _L, _D0), o1.reshape(_B, _L, _D1)
